# baseline (speedup 1.0000x reference)
import functools
import jax
import jax.numpy as jnp
from jax import lax
from jax.experimental import pallas as pl
from jax.experimental.pallas import tpu as pltpu
from jax.experimental.pallas import tpu_sc as plsc

NR, ED, B = 6, 512, 4096
NC, NS = 2, 16
SROWS = 48
BLK = 512

_mesh = plsc.VectorSubcoreMesh(core_axis_name="c", subcore_axis_name="s",
                               num_cores=1)


@functools.partial(
    pl.kernel, mesh=_mesh,
    out_type=jax.ShapeDtypeStruct((NR, ED), jnp.float32),
    scratch_types=[
        pltpu.VMEM((SROWS,), jnp.int32),
        pltpu.VMEM((SROWS, ED), jnp.float32),
        pltpu.SemaphoreType.DMA,
    ],
)
def _sc_gather(table_hbm, idx_hbm, out_hbm, idx_v, rows_v, gsem):
    wid = lax.axis_index("s") * NC + lax.axis_index("c")

    @pl.when(wid == 0)
    def _():
        pltpu.sync_copy(idx_hbm, idx_v)
        pltpu.async_copy(table_hbm.at[idx_v], rows_v, gsem).wait()
        pltpu.make_async_copy(
            rows_v.at[pl.ds(0, NR)], out_hbm, gsem).start()
        pltpu.make_async_copy(
            rows_v.at[pl.ds(0, NR)], out_hbm, gsem).wait()


def _tc_body(t_ref, out_ref):
    out_ref[...] = jnp.broadcast_to(t_ref[...][None], (BLK, NR, ED))


def kernel(token_embed_weight, region_ids, batch_size):
    del batch_size
    idx_rep = jnp.tile(region_ids.astype(jnp.int32), SROWS // NR)
    tokens = _sc_gather(token_embed_weight, idx_rep)
    return pl.pallas_call(
        _tc_body,
        grid=(B // BLK,),
        in_specs=[pl.BlockSpec((NR, ED), lambda i: (0, 0))],
        out_specs=pl.BlockSpec((BLK, NR, ED), lambda i: (i, 0, 0)),
        out_shape=jax.ShapeDtypeStruct((B, NR, ED), jnp.float32),
    )(tokens)
